# baseline (speedup 1.0000x reference)
import functools

import jax
import jax.numpy as jnp
from jax.experimental import pallas as pl
from jax.experimental.pallas import tpu as pltpu

_BM = 2048
_BN = 1024
_NEG = -1e30


def _koleo_body(n, eps, xi_ref, xj_ref, out_ref,
                xi_bf_ref, a2_ref, max_ref):
    j = pl.program_id(0)
    i = pl.program_id(1)
    ncols = pl.num_programs(0)
    rbase = i * _BM

    @pl.when(j == 0)
    def _():
        xi_blk = xi_ref[...]
        xi_bf_ref[pl.ds(rbase, _BM), :] = (-2.0 * xi_blk).astype(jnp.bfloat16)
        a2_ref[pl.ds(rbase, _BM), :] = jnp.sum(
            xi_blk * xi_blk, axis=1, keepdims=True)

    @pl.when((j == 0) & (i == 0))
    def _():
        out_ref[...] = jnp.zeros((1, 1), jnp.float32)

    xj_blk = xj_ref[...]
    ones = jnp.ones((1, xj_blk.shape[1]), jnp.float32)
    b2 = jax.lax.dot_general(
        ones, xj_blk * xj_blk, (((1,), (1,)), ((), ())),
        preferred_element_type=jnp.float32)

    s = jax.lax.dot_general(
        xi_bf_ref[pl.ds(rbase, _BM), :], xj_blk.astype(jnp.bfloat16),
        (((1,), (1,)), ((), ())),
        preferred_element_type=jnp.float32)
    score = s + b2

    rows = rbase + jax.lax.broadcasted_iota(jnp.int32, (_BM, _BN), 0)
    cols = j * _BN + jax.lax.broadcasted_iota(jnp.int32, (_BM, _BN), 1)
    score = jnp.where(rows == cols, _NEG, score)

    m = jnp.max(score, axis=1, keepdims=True)

    @pl.when(j == 0)
    def _():
        max_ref[pl.ds(rbase, _BM), :] = m

    @pl.when(j > 0)
    def _():
        max_ref[pl.ds(rbase, _BM), :] = jnp.maximum(
            max_ref[pl.ds(rbase, _BM), :], m)

    @pl.when(j == ncols - 1)
    def _():
        d2 = a2_ref[pl.ds(rbase, _BM), :] + max_ref[pl.ds(rbase, _BM), :]
        lg = jnp.log(1.0 / (d2 * 0.5 + 1.0) ** 2 + eps)
        out_ref[...] += jnp.sum(lg, keepdims=True)


def kernel(xi, xj):
    eps = 1e-08
    n, k = xi.shape
    nrows = n // _BM

    out = pl.pallas_call(
        functools.partial(_koleo_body, n, eps),
        grid=(n // _BN, nrows),
        in_specs=[
            pl.BlockSpec((_BM, k),
                         lambda j, i: (jnp.where(j == 0, i, nrows - 1), 0)),
            pl.BlockSpec((_BN, k), lambda j, i: (j, 0)),
        ],
        out_specs=pl.BlockSpec((1, 1), lambda j, i: (0, 0)),
        out_shape=jax.ShapeDtypeStruct((1, 1), jnp.float32),
        scratch_shapes=[
            pltpu.VMEM((n, k), jnp.bfloat16),
            pltpu.VMEM((n, 1), jnp.float32),
            pltpu.VMEM((n, 1), jnp.float32),
        ],
        compiler_params=pltpu.CompilerParams(
            dimension_semantics=("arbitrary", "arbitrary")),
    )(xi, xj)
    return out[0, 0] / n
